# baseline (speedup 1.0000x reference)
import functools

import jax
import jax.numpy as jnp
from jax import lax
from jax.experimental import pallas as pl
from jax.experimental.pallas import tpu as pltpu
from jax.experimental.pallas import tpu_sc as plsc

D_MODEL = 64
SCALE = 8.0

_info = plsc.get_sparse_core_info()
NC, NS, L = _info.num_cores, _info.num_subcores, _info.num_lanes
NW = NC * NS

BLK = 128
TP = BLK + 1


@functools.partial(jax.jit, static_argnums=(2, 3))
def _sc_embed(table, xt, n_seq, n_blk):
    mesh = plsc.VectorSubcoreMesh(core_axis_name="c", subcore_axis_name="s")

    @functools.partial(
        pl.kernel,
        mesh=mesh,
        out_type=jax.ShapeDtypeStruct(
            (n_seq, D_MODEL // 8, n_blk, 8, BLK), jnp.float32
        ),
        scratch_types=[
            pltpu.VMEM((n_seq + 2, BLK), jnp.int32),
            pltpu.VMEM((4, BLK, D_MODEL), jnp.float32),
            pltpu.VMEM((2, D_MODEL // 8, 8, TP), jnp.float32),
            pltpu.SemaphoreType.DMA,
            pltpu.SemaphoreType.DMA,
            pltpu.SemaphoreType.DMA,
            pltpu.SemaphoreType.DMA,
            pltpu.SemaphoreType.DMA,
            pltpu.SemaphoreType.DMA,
        ],
        compiler_params=pltpu.CompilerParams(
            use_tc_tiling_on_sc=False, needs_layout_passes=False
        ),
    )
    def k(table2_hbm, xt_hbm, out_hbm, idx_v, rows_v, t_v,
          g0, g1, g2, g3, o0, o1):
        sem_g = (g0, g1, g2, g3)
        sem_o = (o0, o1)
        wid = lax.axis_index("s") * NC + lax.axis_index("c")
        pltpu.sync_copy(
            xt_hbm.at[:, pl.ds(wid * BLK, BLK)], idx_v.at[pl.ds(0, n_seq)]
        )
        iota = lax.iota(jnp.int32, L)
        zeros = iota * 0
        for r in range(2):
            for h in range(BLK // L):
                idx_v[n_seq + r, pl.ds(h * L, L)] = zeros
        dvecs = [iota + q * L for q in range(D_MODEL // L)]
        td_vecs = [lax.shift_right_logical(d, 3) for d in dvecs]
        s_vecs = [lax.bitwise_and(d, 7) for d in dvecs]

        def transpose_block(b, t):
            @plsc.parallel_loop(0, BLK, unroll=16)
            def _tl(l):
                lsplat = zeros + l
                for q in range(D_MODEL // L):
                    v = rows_v[b, l, pl.ds(q * L, L)] * SCALE
                    plsc.store_scatter(
                        t_v.at[t], [td_vecs[q], s_vecs[q], lsplat], v
                    )

        for r in range(2):
            pltpu.async_copy(
                table2_hbm.at[idx_v.at[r]], rows_v.at[r], sem_g[r]
            )

        def step(g, carry):
            for b in range(4):
                j = 4 * g + b
                t = b % 2
                pltpu.async_copy(
                    table2_hbm.at[idx_v.at[j + 2]],
                    rows_v.at[(b + 2) % 4],
                    sem_g[(b + 2) % 4],
                )
                pltpu.make_async_copy(
                    table2_hbm.at[idx_v.at[j]], rows_v.at[b], sem_g[b]
                ).wait()
                @pl.when(j >= 2)
                def _drain():
                    pltpu.make_async_copy(
                        t_v.at[t, :, :, pl.ds(0, BLK)],
                        out_hbm.at[j, :, wid],
                        sem_o[t],
                    ).wait()

                transpose_block(b, t)
                pltpu.async_copy(
                    t_v.at[t, :, :, pl.ds(0, BLK)],
                    out_hbm.at[j, :, wid],
                    sem_o[t],
                )
            return carry

        lax.fori_loop(0, n_seq // 4, step, 0)
        for t in range(2):
            pltpu.make_async_copy(
                t_v.at[t, :, :, pl.ds(0, BLK)],
                out_hbm.at[n_seq - 2 + t, :, wid],
                sem_o[t],
            ).wait()
        for r in range(2):
            pltpu.make_async_copy(
                table2_hbm.at[idx_v.at[n_seq + r]], rows_v.at[r], sem_g[r]
            ).wait()

    return k(table, xt)


def kernel(x, table):
    b, n_seq = x.shape
    assert b == NW * BLK and n_seq % 4 == 0
    xt = x.T.astype(jnp.int32)
    table2 = table
    phys = _sc_embed(table2, xt, n_seq, b // BLK)
    out = phys.transpose(2, 4, 0, 1, 3).reshape(b, n_seq, D_MODEL)
    return out


# speedup vs baseline: 1.0018x; 1.0018x over previous
import functools

import jax
import jax.numpy as jnp
from jax import lax
from jax.experimental import pallas as pl
from jax.experimental.pallas import tpu as pltpu
from jax.experimental.pallas import tpu_sc as plsc

D_MODEL = 64
SCALE = 8.0

_info = plsc.get_sparse_core_info()
NC, NS, L = _info.num_cores, _info.num_subcores, _info.num_lanes
NW = NC * NS

BLK = 128
TP = BLK + 1


@functools.partial(jax.jit, static_argnums=(2, 3))
def _sc_embed(table, xt, n_seq, n_blk):
    mesh = plsc.VectorSubcoreMesh(core_axis_name="c", subcore_axis_name="s")

    @functools.partial(
        pl.kernel,
        mesh=mesh,
        out_type=jax.ShapeDtypeStruct(
            (n_seq, D_MODEL // 8, n_blk, 8, BLK), jnp.float32
        ),
        scratch_types=[
            pltpu.VMEM((n_seq + 2, BLK), jnp.int32),
            pltpu.VMEM((4, BLK, D_MODEL), jnp.float32),
            pltpu.VMEM((2, D_MODEL // 8, 8, TP), jnp.float32),
            pltpu.SemaphoreType.DMA,
            pltpu.SemaphoreType.DMA,
            pltpu.SemaphoreType.DMA,
            pltpu.SemaphoreType.DMA,
            pltpu.SemaphoreType.DMA,
            pltpu.SemaphoreType.DMA,
        ],
        compiler_params=pltpu.CompilerParams(
            use_tc_tiling_on_sc=False, needs_layout_passes=False
        ),
    )
    def k(table2_hbm, xt_hbm, out_hbm, idx_v, rows_v, t_v,
          g0, g1, g2, g3, o0, o1):
        sem_g = (g0, g1, g2, g3)
        sem_o = (o0, o1)
        wid = lax.axis_index("s") * NC + lax.axis_index("c")
        pltpu.sync_copy(
            xt_hbm.at[:, pl.ds(wid * BLK, BLK)], idx_v.at[pl.ds(0, n_seq)]
        )
        iota = lax.iota(jnp.int32, L)
        zeros = iota * 0
        for r in range(2):
            for h in range(BLK // L):
                idx_v[n_seq + r, pl.ds(h * L, L)] = zeros
        dvecs = [iota + q * L for q in range(D_MODEL // L)]
        td_vecs = [lax.shift_right_logical(d, 3) for d in dvecs]
        s_vecs = [lax.bitwise_and(d, 7) for d in dvecs]

        def transpose_block(b, t):
            @plsc.parallel_loop(0, BLK, unroll=8)
            def _tl(l):
                lsplat = zeros + l
                for q in range(D_MODEL // L):
                    v = rows_v[b, l, pl.ds(q * L, L)] * SCALE
                    plsc.store_scatter(
                        t_v.at[t], [td_vecs[q], s_vecs[q], lsplat], v
                    )

        for r in range(2):
            pltpu.async_copy(
                table2_hbm.at[idx_v.at[r]], rows_v.at[r], sem_g[r]
            )

        def step(g, carry):
            for b in range(4):
                j = 4 * g + b
                t = b % 2
                pltpu.async_copy(
                    table2_hbm.at[idx_v.at[j + 2]],
                    rows_v.at[(b + 2) % 4],
                    sem_g[(b + 2) % 4],
                )
                pltpu.make_async_copy(
                    table2_hbm.at[idx_v.at[j]], rows_v.at[b], sem_g[b]
                ).wait()
                @pl.when(j >= 2)
                def _drain():
                    pltpu.make_async_copy(
                        t_v.at[t, :, :, pl.ds(0, BLK)],
                        out_hbm.at[j, :, wid],
                        sem_o[t],
                    ).wait()

                transpose_block(b, t)
                pltpu.async_copy(
                    t_v.at[t, :, :, pl.ds(0, BLK)],
                    out_hbm.at[j, :, wid],
                    sem_o[t],
                )
            return carry

        lax.fori_loop(0, n_seq // 4, step, 0)
        for t in range(2):
            pltpu.make_async_copy(
                t_v.at[t, :, :, pl.ds(0, BLK)],
                out_hbm.at[n_seq - 2 + t, :, wid],
                sem_o[t],
            ).wait()
        for r in range(2):
            pltpu.make_async_copy(
                table2_hbm.at[idx_v.at[n_seq + r]], rows_v.at[r], sem_g[r]
            ).wait()

    return k(table, xt)


def kernel(x, table):
    b, n_seq = x.shape
    assert b == NW * BLK and n_seq % 4 == 0
    xt = x.T.astype(jnp.int32)
    table2 = table
    phys = _sc_embed(table2, xt, n_seq, b // BLK)
    out = phys.transpose(2, 4, 0, 1, 3).reshape(b, n_seq, D_MODEL)
    return out


# speedup vs baseline: 1.0496x; 1.0477x over previous
import functools

import jax
import jax.numpy as jnp
from jax import lax
from jax.experimental import pallas as pl
from jax.experimental.pallas import tpu as pltpu
from jax.experimental.pallas import tpu_sc as plsc

D_MODEL = 64
SCALE = 8.0

_info = plsc.get_sparse_core_info()
NC, NS, L = _info.num_cores, _info.num_subcores, _info.num_lanes
NW = NC * NS

BLK = 128
TP = BLK + 1


@functools.partial(jax.jit, static_argnums=(2, 3))
def _sc_embed(table, xt, n_seq, n_blk):
    mesh = plsc.VectorSubcoreMesh(core_axis_name="c", subcore_axis_name="s")

    @functools.partial(
        pl.kernel,
        mesh=mesh,
        out_type=jax.ShapeDtypeStruct(
            (n_seq, D_MODEL // 8, n_blk, 8, BLK), jnp.float32
        ),
        scratch_types=[
            pltpu.VMEM((n_seq + 1, BLK), jnp.int32),
            pltpu.VMEM((2, BLK, D_MODEL), jnp.float32),
            pltpu.VMEM((2, D_MODEL // 8, 8, TP), jnp.float32),
            pltpu.SemaphoreType.DMA,
            pltpu.SemaphoreType.DMA,
            pltpu.SemaphoreType.DMA,
            pltpu.SemaphoreType.DMA,
        ],
        compiler_params=pltpu.CompilerParams(
            use_tc_tiling_on_sc=False, needs_layout_passes=False
        ),
    )
    def k(table_hbm, xt_hbm, out_hbm, idx_v, rows_v, t_v, g0, g1, o0, o1):
        sem_g = (g0, g1)
        sem_o = (o0, o1)
        wid = lax.axis_index("s") * NC + lax.axis_index("c")
        pltpu.sync_copy(
            xt_hbm.at[:, pl.ds(wid * BLK, BLK)], idx_v.at[pl.ds(0, n_seq)]
        )
        iota = lax.iota(jnp.int32, L)
        zeros = iota * 0
        for h in range(BLK // L):
            idx_v[n_seq, pl.ds(h * L, L)] = zeros
        dvecs = [iota + q * L for q in range(D_MODEL // L)]
        td_vecs = [lax.shift_right_logical(d, 3) for d in dvecs]
        s_vecs = [lax.bitwise_and(d, 7) for d in dvecs]

        def transpose_block(b):
            @plsc.parallel_loop(0, BLK, unroll=8)
            def _tl(l):
                lsplat = zeros + l
                for q in range(D_MODEL // L):
                    v = rows_v[b, l, pl.ds(q * L, L)] * SCALE
                    plsc.store_scatter(
                        t_v.at[b], [td_vecs[q], s_vecs[q], lsplat], v
                    )

        pltpu.async_copy(table_hbm.at[idx_v.at[0]], rows_v.at[0], sem_g[0])

        def step(g, carry):
            for b in range(2):
                j = 2 * g + b
                pltpu.async_copy(
                    table_hbm.at[idx_v.at[j + 1]], rows_v.at[1 - b], sem_g[1 - b]
                )
                pltpu.make_async_copy(
                    table_hbm.at[idx_v.at[j]], rows_v.at[b], sem_g[b]
                ).wait()
                @pl.when(g > 0)
                def _drain():
                    pltpu.make_async_copy(
                        t_v.at[b, :, :, pl.ds(0, BLK)],
                        out_hbm.at[j, :, wid],
                        sem_o[b],
                    ).wait()

                transpose_block(b)
                pltpu.async_copy(
                    t_v.at[b, :, :, pl.ds(0, BLK)],
                    out_hbm.at[j, :, wid],
                    sem_o[b],
                )
            return carry

        lax.fori_loop(0, n_seq // 2, step, 0)
        for b in range(2):
            pltpu.make_async_copy(
                t_v.at[b, :, :, pl.ds(0, BLK)],
                out_hbm.at[n_seq - 2 + b, :, wid],
                sem_o[b],
            ).wait()
        pltpu.make_async_copy(
            table_hbm.at[idx_v.at[n_seq]], rows_v.at[0], sem_g[0]
        ).wait()

    return k(table, xt)


def kernel(x, table):
    b, n_seq = x.shape
    assert b == NW * BLK and n_seq % 2 == 0
    xt = x.T.astype(jnp.int32)
    phys = _sc_embed(table, xt, n_seq, b // BLK)
    out = phys.transpose(2, 4, 0, 1, 3).reshape(b, n_seq, D_MODEL)
    return out


# speedup vs baseline: 1.0519x; 1.0022x over previous
import functools

import jax
import jax.numpy as jnp
from jax import lax
from jax.experimental import pallas as pl
from jax.experimental.pallas import tpu as pltpu
from jax.experimental.pallas import tpu_sc as plsc

D_MODEL = 64
SCALE = 8.0

_info = plsc.get_sparse_core_info()
NC, NS, L = _info.num_cores, _info.num_subcores, _info.num_lanes
NW = NC * NS

BLK = 128
TP = BLK + 1


@functools.partial(jax.jit, static_argnums=(2, 3))
def _sc_embed(table, xt, n_seq, n_blk):
    mesh = plsc.VectorSubcoreMesh(core_axis_name="c", subcore_axis_name="s")

    @functools.partial(
        pl.kernel,
        mesh=mesh,
        out_type=jax.ShapeDtypeStruct(
            (n_seq, D_MODEL // 8, n_blk, 8, BLK), jnp.float32
        ),
        scratch_types=[
            pltpu.VMEM((n_seq + 1, BLK), jnp.int32),
            pltpu.VMEM((2, BLK, D_MODEL), jnp.float32),
            pltpu.VMEM((2, D_MODEL // 8, 8, TP), jnp.float32),
            pltpu.SemaphoreType.DMA,
            pltpu.SemaphoreType.DMA,
            pltpu.SemaphoreType.DMA,
            pltpu.SemaphoreType.DMA,
        ],
        compiler_params=pltpu.CompilerParams(
            use_tc_tiling_on_sc=False, needs_layout_passes=False
        ),
    )
    def k(table_hbm, xt_hbm, out_hbm, idx_v, rows_v, t_v, g0, g1, o0, o1):
        sem_g = (g0, g1)
        sem_o = (o0, o1)
        wid = lax.axis_index("s") * NC + lax.axis_index("c")
        pltpu.sync_copy(
            xt_hbm.at[:, pl.ds(wid * BLK, BLK)], idx_v.at[pl.ds(0, n_seq)]
        )
        iota = lax.iota(jnp.int32, L)
        zeros = iota * 0
        for h in range(BLK // L):
            idx_v[n_seq, pl.ds(h * L, L)] = zeros
        dvecs = [iota + q * L for q in range(D_MODEL // L)]
        td_vecs = [lax.shift_right_logical(d, 3) for d in dvecs]
        s_vecs = [lax.bitwise_and(d, 7) for d in dvecs]

        def transpose_half(b, lo):
            @plsc.parallel_loop(lo, lo + BLK // 2, unroll=8)
            def _tl(l):
                lsplat = zeros + l
                for q in range(D_MODEL // L):
                    v = rows_v[b, l, pl.ds(q * L, L)] * SCALE
                    plsc.store_scatter(
                        t_v.at[b], [td_vecs[q], s_vecs[q], lsplat], v
                    )

        pltpu.async_copy(table_hbm.at[idx_v.at[0]], rows_v.at[0], sem_g[0])

        def step(g, carry):
            for b in range(2):
                j = 2 * g + b
                pltpu.async_copy(
                    table_hbm.at[idx_v.at[j + 1]], rows_v.at[1 - b], sem_g[1 - b]
                )
                pltpu.make_async_copy(
                    table_hbm.at[idx_v.at[j]], rows_v.at[b], sem_g[b]
                ).wait()
                @pl.when(g > 0)
                def _drain():
                    for half in range(2):
                        pltpu.make_async_copy(
                            t_v.at[b, :, :, pl.ds(half * (BLK // 2), BLK // 2)],
                            out_hbm.at[j, :, wid, :, pl.ds(half * (BLK // 2), BLK // 2)],
                            sem_o[b],
                        ).wait()

                transpose_half(b, 0)
                pltpu.async_copy(
                    t_v.at[b, :, :, pl.ds(0, BLK // 2)],
                    out_hbm.at[j, :, wid, :, pl.ds(0, BLK // 2)],
                    sem_o[b],
                )
                transpose_half(b, BLK // 2)
                pltpu.async_copy(
                    t_v.at[b, :, :, pl.ds(BLK // 2, BLK // 2)],
                    out_hbm.at[j, :, wid, :, pl.ds(BLK // 2, BLK // 2)],
                    sem_o[b],
                )
            return carry

        lax.fori_loop(0, n_seq // 2, step, 0)
        for b in range(2):
            for half in range(2):
                pltpu.make_async_copy(
                    t_v.at[b, :, :, pl.ds(half * (BLK // 2), BLK // 2)],
                    out_hbm.at[n_seq - 2 + b, :, wid, :, pl.ds(half * (BLK // 2), BLK // 2)],
                    sem_o[b],
                ).wait()
        pltpu.make_async_copy(
            table_hbm.at[idx_v.at[n_seq]], rows_v.at[0], sem_g[0]
        ).wait()

    return k(table, xt)


def kernel(x, table):
    b, n_seq = x.shape
    assert b == NW * BLK and n_seq % 2 == 0
    xt = x.T.astype(jnp.int32)
    phys = _sc_embed(table, xt, n_seq, b // BLK)
    out = phys.transpose(2, 4, 0, 1, 3).reshape(b, n_seq, D_MODEL)
    return out
